# baseline (speedup 1.0000x reference)
import functools

import jax
import jax.numpy as jnp
from jax import lax
from jax.experimental import pallas as pl
from jax.experimental.pallas import tpu as pltpu
from jax.experimental.pallas import tpu_sc as plsc

_NC = 2
_NS = 16
_LANES = 16

_ZPAD = 8


def _sc_select_gather(table, tid, flag, winner, P, D, V):
    NW = _NC * _NS
    per_w = P // NW
    CH = 64
    NBUF = 10
    n_iter = per_w // CH
    n_round = n_iter // NBUF
    assert n_iter % NBUF == 0
    zero_idx = V
    unk_base = V + _ZPAD - 1
    mesh = plsc.VectorSubcoreMesh(core_axis_name="c", subcore_axis_name="s")

    @functools.partial(
        pl.kernel,
        mesh=mesh,
        out_type=jax.ShapeDtypeStruct((P, D), jnp.float32),
        scratch_types=[
            pltpu.VMEM((per_w,), jnp.int32),
            pltpu.VMEM((per_w,), jnp.int32),
            pltpu.VMEM((per_w,), jnp.int32),
            pltpu.VMEM((per_w,), jnp.int32),
        ] + [pltpu.VMEM((CH, D), jnp.float32) for _ in range(NBUF)]
          + [pltpu.SemaphoreType.DMA for _ in range(2 * NBUF)],
    )
    def k(table_hbm, tid_hbm, flag_hbm, w_hbm, out_hbm,
          tid_v, flag_v, w_v, idx_v, *bufs_and_sems):
        rows = bufs_and_sems[:NBUF]
        semg = bufs_and_sems[NBUF:2 * NBUF]
        semw = bufs_and_sems[2 * NBUF:]
        wid = lax.axis_index("s") * _NC + lax.axis_index("c")
        base = wid * per_w

        pltpu.sync_copy(tid_hbm.at[pl.ds(base, per_w)], tid_v)
        pltpu.sync_copy(flag_hbm.at[pl.ds(base, per_w)], flag_v)
        pltpu.sync_copy(w_hbm.at[pl.ds(base, per_w)], w_v)

        @pl.loop(0, per_w // _LANES)
        def _(j):
            sl = pl.ds(j * _LANES, _LANES)
            t = tid_v[sl]
            f = flag_v[sl]
            w = w_v[sl]
            cached = jnp.where(f > 0, t, zero_idx)
            idx_v[sl] = jnp.where(w > 0, w + unk_base, cached)

        def start_gather(i, b):
            pltpu.async_copy(
                table_hbm.at[idx_v.at[pl.ds(i * CH, CH)]], rows[b], semg[b])

        def wait_gather(i, b):
            pltpu.make_async_copy(
                table_hbm.at[idx_v.at[pl.ds(i * CH, CH)]], rows[b],
                semg[b]).wait()

        def start_wb(i, b):
            pltpu.async_copy(
                rows[b], out_hbm.at[pl.ds(base + i * CH, CH)], semw[b])

        def wait_wb(i, b):
            pltpu.make_async_copy(
                rows[b], out_hbm.at[pl.ds(base + i * CH, CH)], semw[b]).wait()

        for b in range(NBUF):
            start_gather(b, b)

        @pl.loop(0, n_iter, step=NBUF)
        def _(g):
            for b in range(NBUF):
                i = g + b
                wait_gather(i, b)
                start_wb(i, b)

                @pl.when(i + NBUF < n_iter)
                def _():
                    wait_wb(i, b)
                    start_gather(i + NBUF, b)

        for b in range(NBUF):
            wait_wb(n_iter - NBUF + b, b)

    return k(table, tid, flag, winner)


def _tc_mask(seq_lens, L):
    B = seq_lens.shape[0]

    def body(sl_ref, o_ref):
        col = lax.broadcasted_iota(jnp.int32, (B, L), 1)
        o_ref[...] = (col < sl_ref[...]).astype(jnp.float32)

    return pl.pallas_call(
        body,
        out_shape=jax.ShapeDtypeStruct((B, L), jnp.float32),
    )(seq_lens.reshape(B, 1))


def kernel(cache_table, unknown_embeddings, token_ids, known_flag,
           unknown_rows, unknown_cols, seq_lens):
    B, L = token_ids.shape
    V, D = cache_table.shape
    U = unknown_embeddings.shape[0]
    P = B * L

    table = jnp.concatenate(
        [cache_table, jnp.zeros((_ZPAD, D), jnp.float32), unknown_embeddings],
        axis=0)
    lin = unknown_rows.astype(jnp.int32) * L + unknown_cols.astype(jnp.int32)
    winner = jnp.zeros((P,), jnp.int32).at[lin].max(
        jnp.arange(1, U + 1, dtype=jnp.int32))

    emb = _sc_select_gather(table, token_ids.reshape(P), known_flag.reshape(P),
                            winner, P, D, V)
    mask = _tc_mask(seq_lens, L)
    return emb.reshape(B, L, D), mask


# speedup vs baseline: 9.7119x; 9.7119x over previous
import functools

import jax
import jax.numpy as jnp
from jax import lax
from jax.experimental import pallas as pl
from jax.experimental.pallas import tpu as pltpu
from jax.experimental.pallas import tpu_sc as plsc

_NC = 2
_NS = 16
_LANES = 16

_ZPAD = 128


def _sc_select_gather(table, tid, flag, winner, P, D, V):
    NW = _NC * _NS
    per_w = P // NW
    CH = 64
    NBUF = 10
    n_iter = per_w // CH
    n_round = n_iter // NBUF
    assert n_iter % NBUF == 0
    zero_idx = V
    unk_base = V + _ZPAD - 1
    mesh = plsc.VectorSubcoreMesh(core_axis_name="c", subcore_axis_name="s")

    @functools.partial(
        pl.kernel,
        mesh=mesh,
        out_type=jax.ShapeDtypeStruct((P, D), jnp.float32),
        scratch_types=[
            pltpu.VMEM((per_w,), jnp.int32),
            pltpu.VMEM((per_w,), jnp.int32),
            pltpu.VMEM((per_w,), jnp.int32),
            pltpu.VMEM((per_w,), jnp.int32),
        ] + [pltpu.VMEM((CH, D), jnp.float32) for _ in range(NBUF)]
          + [pltpu.SemaphoreType.DMA for _ in range(2 * NBUF)],
    )
    def k(table_hbm, tid_hbm, flag_hbm, w_hbm, out_hbm,
          tid_v, flag_v, w_v, idx_v, *bufs_and_sems):
        rows = bufs_and_sems[:NBUF]
        semg = bufs_and_sems[NBUF:2 * NBUF]
        semw = bufs_and_sems[2 * NBUF:]
        wid = lax.axis_index("s") * _NC + lax.axis_index("c")
        base = wid * per_w

        pltpu.sync_copy(tid_hbm.at[pl.ds(base, per_w)], tid_v)
        pltpu.sync_copy(flag_hbm.at[pl.ds(base, per_w)], flag_v)
        pltpu.sync_copy(w_hbm.at[pl.ds(base, per_w)], w_v)

        lane = lax.iota(jnp.int32, _LANES)

        @pl.loop(0, per_w // _LANES)
        def _(j):
            sl = pl.ds(j * _LANES, _LANES)
            t = tid_v[sl]
            f = flag_v[sl]
            w = w_v[sl]
            zrow = zero_idx + jnp.bitwise_and(j * _LANES + lane, _ZPAD - 1)
            cached = jnp.where(f > 0, t, zrow)
            idx_v[sl] = jnp.where(w > 0, w + unk_base, cached)

        def start_gather(i, b):
            pltpu.async_copy(
                table_hbm.at[idx_v.at[pl.ds(i * CH, CH)]], rows[b], semg[b])

        def wait_gather(i, b):
            pltpu.make_async_copy(
                table_hbm.at[idx_v.at[pl.ds(i * CH, CH)]], rows[b],
                semg[b]).wait()

        def start_wb(i, b):
            pltpu.async_copy(
                rows[b], out_hbm.at[pl.ds(base + i * CH, CH)], semw[b])

        def wait_wb(i, b):
            pltpu.make_async_copy(
                rows[b], out_hbm.at[pl.ds(base + i * CH, CH)], semw[b]).wait()

        for b in range(NBUF):
            start_gather(b, b)

        @pl.loop(0, n_iter, step=NBUF)
        def _(g):
            for b in range(NBUF):
                i = g + b
                wait_gather(i, b)
                start_wb(i, b)

                @pl.when(i + NBUF < n_iter)
                def _():
                    wait_wb(i, b)
                    start_gather(i + NBUF, b)

        for b in range(NBUF):
            wait_wb(n_iter - NBUF + b, b)

    return k(table, tid, flag, winner)


def _tc_mask(seq_lens, L):
    B = seq_lens.shape[0]

    def body(sl_ref, o_ref):
        col = lax.broadcasted_iota(jnp.int32, (B, L), 1)
        o_ref[...] = (col < sl_ref[...]).astype(jnp.float32)

    return pl.pallas_call(
        body,
        out_shape=jax.ShapeDtypeStruct((B, L), jnp.float32),
    )(seq_lens.reshape(B, 1))


def kernel(cache_table, unknown_embeddings, token_ids, known_flag,
           unknown_rows, unknown_cols, seq_lens):
    B, L = token_ids.shape
    V, D = cache_table.shape
    U = unknown_embeddings.shape[0]
    P = B * L

    table = jnp.concatenate(
        [cache_table, jnp.zeros((_ZPAD, D), jnp.float32), unknown_embeddings],
        axis=0)
    lin = unknown_rows.astype(jnp.int32) * L + unknown_cols.astype(jnp.int32)
    winner = jnp.zeros((P,), jnp.int32).at[lin].max(
        jnp.arange(1, U + 1, dtype=jnp.int32))

    emb = _sc_select_gather(table, token_ids.reshape(P), known_flag.reshape(P),
                            winner, P, D, V)
    mask = _tc_mask(seq_lens, L)
    return emb.reshape(B, L, D), mask


# speedup vs baseline: 9.7360x; 1.0025x over previous
import functools

import jax
import jax.numpy as jnp
from jax import lax
from jax.experimental import pallas as pl
from jax.experimental.pallas import tpu as pltpu
from jax.experimental.pallas import tpu_sc as plsc

_NC = 2
_NS = 16
_LANES = 16

_ZPAD = 128


def _sc_select_gather(table, tid, flag, winner, P, D, V):
    NW = _NC * _NS
    per_w = P // NW
    CH = 128
    NBUF = 5
    n_iter = per_w // CH
    n_round = n_iter // NBUF
    assert n_iter % NBUF == 0
    zero_idx = V
    unk_base = V + _ZPAD - 1
    mesh = plsc.VectorSubcoreMesh(core_axis_name="c", subcore_axis_name="s")

    @functools.partial(
        pl.kernel,
        mesh=mesh,
        out_type=jax.ShapeDtypeStruct((P, D), jnp.float32),
        scratch_types=[
            pltpu.VMEM((per_w,), jnp.int32),
            pltpu.VMEM((per_w,), jnp.int32),
            pltpu.VMEM((per_w,), jnp.int32),
            pltpu.VMEM((per_w,), jnp.int32),
        ] + [pltpu.VMEM((CH, D), jnp.float32) for _ in range(NBUF)]
          + [pltpu.SemaphoreType.DMA for _ in range(2 * NBUF)],
    )
    def k(table_hbm, tid_hbm, flag_hbm, w_hbm, out_hbm,
          tid_v, flag_v, w_v, idx_v, *bufs_and_sems):
        rows = bufs_and_sems[:NBUF]
        semg = bufs_and_sems[NBUF:2 * NBUF]
        semw = bufs_and_sems[2 * NBUF:]
        wid = lax.axis_index("s") * _NC + lax.axis_index("c")
        base = wid * per_w

        pltpu.sync_copy(tid_hbm.at[pl.ds(base, per_w)], tid_v)
        pltpu.sync_copy(flag_hbm.at[pl.ds(base, per_w)], flag_v)
        pltpu.sync_copy(w_hbm.at[pl.ds(base, per_w)], w_v)

        lane = lax.iota(jnp.int32, _LANES)

        @pl.loop(0, per_w // _LANES)
        def _(j):
            sl = pl.ds(j * _LANES, _LANES)
            t = tid_v[sl]
            f = flag_v[sl]
            w = w_v[sl]
            zrow = zero_idx + jnp.bitwise_and(j * _LANES + lane, _ZPAD - 1)
            cached = jnp.where(f > 0, t, zrow)
            idx_v[sl] = jnp.where(w > 0, w + unk_base, cached)

        def start_gather(i, b):
            pltpu.async_copy(
                table_hbm.at[idx_v.at[pl.ds(i * CH, CH)]], rows[b], semg[b])

        def wait_gather(i, b):
            pltpu.make_async_copy(
                table_hbm.at[idx_v.at[pl.ds(i * CH, CH)]], rows[b],
                semg[b]).wait()

        def start_wb(i, b):
            pltpu.async_copy(
                rows[b], out_hbm.at[pl.ds(base + i * CH, CH)], semw[b])

        def wait_wb(i, b):
            pltpu.make_async_copy(
                rows[b], out_hbm.at[pl.ds(base + i * CH, CH)], semw[b]).wait()

        for b in range(NBUF):
            start_gather(b, b)

        @pl.loop(0, n_iter, step=NBUF)
        def _(g):
            for b in range(NBUF):
                i = g + b
                wait_gather(i, b)
                start_wb(i, b)

                @pl.when(i + NBUF < n_iter)
                def _():
                    wait_wb(i, b)
                    start_gather(i + NBUF, b)

        for b in range(NBUF):
            wait_wb(n_iter - NBUF + b, b)

    return k(table, tid, flag, winner)


def _tc_mask(seq_lens, L):
    B = seq_lens.shape[0]

    def body(sl_ref, o_ref):
        col = lax.broadcasted_iota(jnp.int32, (B, L), 1)
        o_ref[...] = (col < sl_ref[...]).astype(jnp.float32)

    return pl.pallas_call(
        body,
        out_shape=jax.ShapeDtypeStruct((B, L), jnp.float32),
    )(seq_lens.reshape(B, 1))


def kernel(cache_table, unknown_embeddings, token_ids, known_flag,
           unknown_rows, unknown_cols, seq_lens):
    B, L = token_ids.shape
    V, D = cache_table.shape
    U = unknown_embeddings.shape[0]
    P = B * L

    table = jnp.concatenate(
        [cache_table, jnp.zeros((_ZPAD, D), jnp.float32), unknown_embeddings],
        axis=0)
    lin = unknown_rows.astype(jnp.int32) * L + unknown_cols.astype(jnp.int32)
    winner = jnp.zeros((P,), jnp.int32).at[lin].max(
        jnp.arange(1, U + 1, dtype=jnp.int32))

    emb = _sc_select_gather(table, token_ids.reshape(P), known_flag.reshape(P),
                            winner, P, D, V)
    mask = _tc_mask(seq_lens, L)
    return emb.reshape(B, L, D), mask
